# Initial kernel scaffold; baseline (speedup 1.0000x reference)
import jax
import jax.numpy as jnp
from jax.experimental import pallas as pl


def kernel(e_feat, m_feat, edge_index, params):
    raise NotImplementedError("write your pallas kernel here")


# speedup vs baseline: 3.8633x; 3.8633x over previous
import functools

import jax
import jax.numpy as jnp
from jax import lax
from jax.experimental import pallas as pl
from jax.experimental.pallas import tpu as pltpu
import jax.experimental.pallas.tpu_sc as plsc

N_E = 16
N_M = 9984
N = 10000
NPAD = 10240
E = 160000
HID = 256
NC, NS = 2, 16
CHUNK = 128
NCHUNK = E // CHUNK
TPT = (NCHUNK + NS - 1) // NS


def _sc_degrees(edge3):
    mesh = plsc.VectorSubcoreMesh(core_axis_name="c", subcore_axis_name="s")

    @functools.partial(
        pl.kernel,
        out_type=jax.ShapeDtypeStruct((NC, NPAD), jnp.float32),
        mesh=mesh,
        scratch_types=[
            pltpu.VMEM_SHARED((NPAD,), jnp.float32),
            pltpu.VMEM((CHUNK,), jnp.int32),
            pltpu.VMEM((CHUNK,), jnp.float32),
            pltpu.VMEM((640,), jnp.float32),
        ],
    )
    def k(edge_hbm, out_hbm, hist_sh, idx_v, ones_v, z_v):
        c = lax.axis_index("c")
        s = lax.axis_index("s")
        for i in range(CHUNK // 16):
            ones_v[pl.ds(i * 16, 16)] = jnp.full((16,), 1.0, jnp.float32)
        for i in range(640 // 16):
            z_v[pl.ds(i * 16, 16)] = jnp.zeros((16,), jnp.float32)
        pltpu.sync_copy(z_v, hist_sh.at[pl.ds(s * 640, 640)])
        plsc.subcore_barrier()

        def body(t, carry):
            cj = s + NS * t

            @pl.when(cj < NCHUNK)
            def _():
                pltpu.sync_copy(edge_hbm.at[c, cj], idx_v)
                pltpu.sync_copy(ones_v, hist_sh.at[idx_v], add=True)

            return carry

        lax.fori_loop(0, TPT, body, 0)
        plsc.subcore_barrier()
        pltpu.sync_copy(hist_sh.at[pl.ds(s * 640, 640)],
                        out_hbm.at[c, pl.ds(s * 640, 640)])

    return k(edge3)


def _sc_aggregate(table, edge3, zeros, fh):
    mesh = plsc.VectorSubcoreMesh(core_axis_name="c", subcore_axis_name="s")

    @functools.partial(
        pl.kernel,
        out_type=jax.ShapeDtypeStruct((NC, NPAD, fh), jnp.float32),
        mesh=mesh,
        scratch_types=[
            pltpu.VMEM_SHARED((NPAD, fh), jnp.float32),
            pltpu.VMEM((CHUNK,), jnp.int32),
            pltpu.VMEM((CHUNK,), jnp.int32),
            pltpu.VMEM((CHUNK,), jnp.int32),
            pltpu.VMEM((CHUNK, fh), jnp.float32),
            pltpu.SemaphoreType.DMA,
        ],
    )
    def k(table_hbm, edge_hbm, zero_hbm, out_hbm,
          acc_sh, src_v, srco_v, dst_v, rows_v, sem):
        c = lax.axis_index("c")
        s = lax.axis_index("s")
        pltpu.sync_copy(zero_hbm.at[pl.ds(s * 640, 640)],
                        acc_sh.at[pl.ds(s * 640, 640)])
        plsc.subcore_barrier()
        off = c * N

        def body(t, carry):
            cj = s + NS * t

            @pl.when(cj < NCHUNK)
            def _():
                pltpu.sync_copy(edge_hbm.at[0, cj], src_v)
                pltpu.sync_copy(edge_hbm.at[1, cj], dst_v)
                for i in range(CHUNK // 16):
                    srco_v[pl.ds(i * 16, 16)] = src_v[pl.ds(i * 16, 16)] + off
                pltpu.async_copy(table_hbm.at[srco_v], rows_v, sem).wait()
                pltpu.sync_copy(rows_v, acc_sh.at[dst_v], add=True)

            return carry

        lax.fori_loop(0, TPT, body, 0)
        plsc.subcore_barrier()
        pltpu.sync_copy(acc_sh.at[pl.ds(s * 640, 640)],
                        out_hbm.at[c, pl.ds(s * 640, 640)])

    return k(table, edge3, zeros)


def _lrelu(x):
    return jnp.where(x >= 0, x, 0.1 * x)


def _tc_conv1(e64m, w1, beta1):
    def body(w_ref, b_ref, x_ref, o_ref):
        acc = jnp.zeros((400, 400), jnp.float32)
        for a in range(4):
            for b in range(4):
                xs = x_ref[b].reshape(400, 4, 400)[:, a, :]
                acc = acc + xs * w_ref[a, b]
        o_ref[0] = _lrelu(acc + b_ref[0, 0])

    return pl.pallas_call(
        body,
        grid=(16,),
        in_specs=[
            pl.BlockSpec(memory_space=pltpu.SMEM),
            pl.BlockSpec(memory_space=pltpu.SMEM),
            pl.BlockSpec((4, 1600, 400), lambda u: (u, 0, 0)),
        ],
        out_specs=pl.BlockSpec((1, 400, 400), lambda u: (u, 0, 0)),
        out_shape=jax.ShapeDtypeStruct((16, 400, 400), jnp.float32),
    )(w1, beta1, e64m)


def _tc_head(out1d, w2, beta2, w3, beta3, wp, fce_b, deg_e):
    def body(w2_ref, b2_ref, w3_ref, b3_ref, x_ref, wp_ref, fb_ref, de_ref,
             o_ref):
        out2 = []
        for u2 in range(4):
            acc = jnp.zeros((100, 400), jnp.float32)
            for a in range(4):
                for b in range(4):
                    xs = x_ref[4 * u2 + b].reshape(100, 4, 400)[:, a, :]
                    acc = acc + xs * w2_ref[a, b]
            out2.append(_lrelu(acc + b2_ref[0, 0]))
        acc3 = jnp.zeros((25, 400), jnp.float32)
        for a in range(4):
            for b in range(4):
                xs = out2[b].reshape(25, 4, 400)[:, a, :]
                acc3 = acc3 + xs * w3_ref[a, b]
        out3 = _lrelu(acc3 + b3_ref[0, 0])
        o3r = out3.T.reshape(16, 25, 25)
        he = jnp.zeros((16, HID), jnp.float32)
        for t3 in range(25):
            he = he + jnp.dot(o3r[:, t3, :], wp_ref[t3],
                              preferred_element_type=jnp.float32)
        dinv = lax.rsqrt(jnp.maximum(de_ref[...], 1.0))
        he = (he + fb_ref[...]) * dinv
        o_ref[0] = he[:, 0:128]
        o_ref[1] = he[:, 128:256]

    return pl.pallas_call(
        body,
        in_specs=[
            pl.BlockSpec(memory_space=pltpu.SMEM),
            pl.BlockSpec(memory_space=pltpu.SMEM),
            pl.BlockSpec(memory_space=pltpu.SMEM),
            pl.BlockSpec(memory_space=pltpu.SMEM),
            pl.BlockSpec((16, 400, 400), lambda: (0, 0, 0)),
            pl.BlockSpec((25, 25, HID), lambda: (0, 0, 0)),
            pl.BlockSpec((1, HID), lambda: (0, 0)),
            pl.BlockSpec((16, 1), lambda: (0, 0)),
        ],
        out_specs=pl.BlockSpec((2, 16, 128), lambda: (0, 0, 0)),
        out_shape=jax.ShapeDtypeStruct((2, 16, 128), jnp.float32),
    )(w2, beta2, w3, beta3, out1d, wp, fce_b, deg_e)


def _tc_fcm(m_feat, wm, bm, deg_m):
    def body(x_ref, w_ref, b_ref, d_ref, o_ref):
        z = jnp.dot(x_ref[...], w_ref[...],
                    preferred_element_type=jnp.float32) + b_ref[...]
        dinv = lax.rsqrt(jnp.maximum(d_ref[...], 1.0))
        o_ref[0] = z * dinv

    return pl.pallas_call(
        body,
        grid=(2, 16),
        in_specs=[
            pl.BlockSpec((624, HID), lambda h, r: (r, 0)),
            pl.BlockSpec((HID, 128), lambda h, r: (0, h)),
            pl.BlockSpec((1, 128), lambda h, r: (0, h)),
            pl.BlockSpec((624, 1), lambda h, r: (r, 0)),
        ],
        out_specs=pl.BlockSpec((1, 624, 128), lambda h, r: (h, r, 0)),
        out_shape=jax.ShapeDtypeStruct((2, N_M, 128), jnp.float32),
    )(m_feat, wm, bm, deg_m)


def _tc_layer(agg, deg_in, deg_out, wr, b):
    def body(a_ref, di_ref, do_ref, w_ref, b_ref, o_ref):
        din = lax.rsqrt(jnp.maximum(di_ref[...], 1.0))
        a = a_ref[...]
        z = (jnp.dot(a[0] * din, w_ref[0], preferred_element_type=jnp.float32)
             + jnp.dot(a[1] * din, w_ref[1], preferred_element_type=jnp.float32)
             + b_ref[...])
        dout = lax.rsqrt(jnp.maximum(do_ref[...], 1.0))
        o_ref[0] = jnp.maximum(z, 0.0) * dout

    return pl.pallas_call(
        body,
        grid=(2, 25),
        in_specs=[
            pl.BlockSpec((2, 400, 128), lambda h, r: (0, r, 0)),
            pl.BlockSpec((400, 1), lambda h, r: (r, 0)),
            pl.BlockSpec((400, 1), lambda h, r: (r, 0)),
            pl.BlockSpec((2, 128, 128), lambda h, r: (0, 0, h)),
            pl.BlockSpec((1, 128), lambda h, r: (0, h)),
        ],
        out_specs=pl.BlockSpec((1, 400, 128), lambda h, r: (h, r, 0)),
        out_shape=jax.ShapeDtypeStruct((2, N, 128), jnp.float32),
    )(agg, deg_in, deg_out, wr, b)


def _tc_final(agg, deg_in, w3r, b3):
    def body(a_ref, di_ref, w3_ref, b_ref, o_ref):
        din = lax.rsqrt(jnp.maximum(di_ref[...], 1.0))
        a = a_ref[...]
        o_ref[...] = (
            jnp.dot(a[0] * din, w3_ref[0], preferred_element_type=jnp.float32)
            + jnp.dot(a[1] * din, w3_ref[1], preferred_element_type=jnp.float32)
            + b_ref[...])

    return pl.pallas_call(
        body,
        grid=(25,),
        in_specs=[
            pl.BlockSpec((2, 400, 128), lambda r: (0, r, 0)),
            pl.BlockSpec((400, 1), lambda r: (r, 0)),
            pl.BlockSpec((2, 128, 16), lambda r: (0, 0, 0)),
            pl.BlockSpec((1, 16), lambda r: (0, 0)),
        ],
        out_specs=pl.BlockSpec((400, 16), lambda r: (r, 0)),
        out_shape=jax.ShapeDtypeStruct((N, 16), jnp.float32),
    )(agg, deg_in, w3r, b3)


def kernel(e_feat, m_feat, edge_index, params):
    p = params
    gains = [p['bn_gamma'][i][0] / jnp.sqrt(jnp.float32(1.0 + 1e-5))
             for i in range(3)]
    w1 = p['conv_w'][0][0, 0] * gains[0]
    w2 = p['conv_w'][1][0, 0] * gains[1]
    w3 = p['conv_w'][2][0, 0] * gains[2]
    b1 = p['bn_beta'][0].reshape(1, 1)
    b2 = p['bn_beta'][1].reshape(1, 1)
    b3 = p['bn_beta'][2].reshape(1, 1)

    e64m = (e_feat.reshape(16, 1600, 25, 64)
            .transpose(3, 1, 0, 2).reshape(64, 1600, 400))
    edge3 = edge_index.reshape(2, NCHUNK, CHUNK)

    deg = _sc_degrees(edge3)
    deg_out = deg[0].reshape(NPAD, 1)
    deg_in = deg[1].reshape(NPAD, 1)

    out1d = _tc_conv1(e64m, w1, b1)
    wp = p['fc_e_W'].reshape(25, 25, HID).transpose(1, 0, 2)
    he2 = _tc_head(out1d, w2, b2, w3, b3, wp,
                   p['fc_e_b'].reshape(1, HID), deg_out[0:N_E])
    hm2 = _tc_fcm(m_feat, p['fc_m_W'], p['fc_m_b'].reshape(1, HID),
                  deg_out[N_E:N])
    hn0 = jnp.concatenate([he2, hm2], axis=1)

    z128 = jnp.zeros((NPAD, 128), jnp.float32)

    agg1 = _sc_aggregate(hn0.reshape(2 * N, 128), edge3, z128, 128)
    hn1 = _tc_layer(agg1, deg_in, deg_out,
                    p['gc_W'][0].reshape(2, 128, HID), p['gc_b'][0].reshape(1, HID))
    agg2 = _sc_aggregate(hn1.reshape(2 * N, 128), edge3, z128, 128)
    hn2 = _tc_layer(agg2, deg_in, deg_out,
                    p['gc_W'][1].reshape(2, 128, HID), p['gc_b'][1].reshape(1, HID))
    agg3 = _sc_aggregate(hn2.reshape(2 * N, 128), edge3, z128, 128)
    return _tc_final(agg3, deg_in, p['gc_W'][2].reshape(2, 128, 16),
                     p['gc_b'][2].reshape(1, 16))


# speedup vs baseline: 5.9976x; 1.5525x over previous
import functools

import jax
import jax.numpy as jnp
from jax import lax
from jax.experimental import pallas as pl
from jax.experimental.pallas import tpu as pltpu
import jax.experimental.pallas.tpu_sc as plsc

N_E = 16
N_M = 9984
N = 10000
NPAD = 10240
E = 160000
HID = 256
NC, NS = 2, 16
CHUNK = 128
NCHUNK = E // CHUNK
TPT = (NCHUNK + NS - 1) // NS


def _sc_degrees(edge3):
    mesh = plsc.VectorSubcoreMesh(core_axis_name="c", subcore_axis_name="s")

    @functools.partial(
        pl.kernel,
        out_type=jax.ShapeDtypeStruct((NC, NPAD), jnp.float32),
        mesh=mesh,
        scratch_types=[
            pltpu.VMEM_SHARED((NPAD,), jnp.float32),
            pltpu.VMEM((CHUNK,), jnp.int32),
            pltpu.VMEM((CHUNK,), jnp.float32),
            pltpu.VMEM((640,), jnp.float32),
        ],
    )
    def k(edge_hbm, out_hbm, hist_sh, idx_v, ones_v, z_v):
        c = lax.axis_index("c")
        s = lax.axis_index("s")
        for i in range(CHUNK // 16):
            ones_v[pl.ds(i * 16, 16)] = jnp.full((16,), 1.0, jnp.float32)
        for i in range(640 // 16):
            z_v[pl.ds(i * 16, 16)] = jnp.zeros((16,), jnp.float32)
        pltpu.sync_copy(z_v, hist_sh.at[pl.ds(s * 640, 640)])
        plsc.subcore_barrier()

        def body(t, carry):
            cj = s + NS * t

            @pl.when(cj < NCHUNK)
            def _():
                pltpu.sync_copy(edge_hbm.at[c, cj], idx_v)
                pltpu.sync_copy(ones_v, hist_sh.at[idx_v], add=True)

            return carry

        lax.fori_loop(0, TPT, body, 0)
        plsc.subcore_barrier()
        pltpu.sync_copy(hist_sh.at[pl.ds(s * 640, 640)],
                        out_hbm.at[c, pl.ds(s * 640, 640)])

    return k(edge3)


def _sc_aggregate(table, edge3, zeros, fh):
    mesh = plsc.VectorSubcoreMesh(core_axis_name="c", subcore_axis_name="s")

    @functools.partial(
        pl.kernel,
        out_type=jax.ShapeDtypeStruct((NC, NPAD, fh), jnp.float32),
        mesh=mesh,
        scratch_types=[
            pltpu.VMEM_SHARED((NPAD, fh), jnp.float32),
            pltpu.VMEM((CHUNK,), jnp.int32),
            pltpu.VMEM((CHUNK,), jnp.int32),
            pltpu.VMEM((CHUNK,), jnp.int32),
            pltpu.VMEM((CHUNK, fh), jnp.float32),
            pltpu.SemaphoreType.DMA,
        ],
    )
    def k(table_hbm, edge_hbm, zero_hbm, out_hbm,
          acc_sh, src_v, srco_v, dst_v, rows_v, sem):
        c = lax.axis_index("c")
        s = lax.axis_index("s")
        pltpu.sync_copy(zero_hbm.at[pl.ds(s * 640, 640)],
                        acc_sh.at[pl.ds(s * 640, 640)])
        plsc.subcore_barrier()
        off = c * N

        def body(t, carry):
            cj = s + NS * t

            @pl.when(cj < NCHUNK)
            def _():
                pltpu.sync_copy(edge_hbm.at[0, cj], src_v)
                pltpu.sync_copy(edge_hbm.at[1, cj], dst_v)
                for i in range(CHUNK // 16):
                    srco_v[pl.ds(i * 16, 16)] = src_v[pl.ds(i * 16, 16)] + off
                pltpu.async_copy(table_hbm.at[srco_v], rows_v, sem).wait()
                pltpu.sync_copy(rows_v, acc_sh.at[dst_v], add=True)

            return carry

        lax.fori_loop(0, TPT, body, 0)
        plsc.subcore_barrier()
        pltpu.sync_copy(acc_sh.at[pl.ds(s * 640, 640)],
                        out_hbm.at[c, pl.ds(s * 640, 640)])

    return k(table, edge3, zeros)


def _lrelu(x):
    return jnp.where(x >= 0, x, 0.1 * x)


import numpy as _np

_JMAP = (16 * (_np.arange(400) % 25) + _np.arange(400) // 25)
_C1MASK = (_np.arange(1600)[:, None] // 4 == _JMAP[None, :]).astype(_np.float32)
_C1BSEL = _np.arange(1600) % 4


def _conv1_matrix(w1):
    cols = [ _C1MASK * w1[a][_C1BSEL][:, None] for a in range(4) ]
    return jnp.concatenate(cols, axis=1).astype(jnp.bfloat16)


def _tc_conv1(e_feat, m1, beta1):
    def body(b_ref, x_ref, m_ref, o_ref):
        xb = x_ref[0].astype(jnp.bfloat16)
        v = jnp.dot(xb, m_ref[...], preferred_element_type=jnp.float32)
        vr = v.reshape(400, 4, 1600)
        acc = (vr[:, 0, 0:400] + vr[:, 1, 400:800]
               + vr[:, 2, 800:1200] + vr[:, 3, 1200:1600])
        o_ref[0] = _lrelu(acc + b_ref[0, 0])

    return pl.pallas_call(
        body,
        grid=(16,),
        in_specs=[
            pl.BlockSpec(memory_space=pltpu.SMEM),
            pl.BlockSpec((1, 1600, 1600), lambda img: (img, 0, 0)),
            pl.BlockSpec((1600, 1600), lambda img: (0, 0)),
        ],
        out_specs=pl.BlockSpec((1, 400, 400), lambda img: (img, 0, 0)),
        out_shape=jax.ShapeDtypeStruct((16, 400, 400), jnp.float32),
    )(beta1, e_feat, m1)


def _tc_head(out1d, w2, beta2, w3, beta3, wp, fce_b, deg_e):
    def body(w2_ref, b2_ref, w3_ref, b3_ref, x_ref, wp_ref, fb_ref, de_ref,
             o_ref):
        out2 = []
        for u2 in range(4):
            acc = jnp.zeros((100, 400), jnp.float32)
            for a in range(4):
                for b in range(4):
                    xs = x_ref[4 * u2 + b].reshape(100, 4, 400)[:, a, :]
                    acc = acc + xs * w2_ref[a, b]
            out2.append(_lrelu(acc + b2_ref[0, 0]))
        acc3 = jnp.zeros((25, 400), jnp.float32)
        for a in range(4):
            for b in range(4):
                xs = out2[b].reshape(25, 4, 400)[:, a, :]
                acc3 = acc3 + xs * w3_ref[a, b]
        out3 = _lrelu(acc3 + b3_ref[0, 0])
        o3r = out3.T.reshape(16, 25, 25)
        he = jnp.zeros((16, HID), jnp.float32)
        for t3 in range(25):
            he = he + jnp.dot(o3r[:, t3, :], wp_ref[t3],
                              preferred_element_type=jnp.float32)
        dinv = lax.rsqrt(jnp.maximum(de_ref[...], 1.0))
        he = (he + fb_ref[...]) * dinv
        o_ref[0] = he[:, 0:128]
        o_ref[1] = he[:, 128:256]

    return pl.pallas_call(
        body,
        in_specs=[
            pl.BlockSpec(memory_space=pltpu.SMEM),
            pl.BlockSpec(memory_space=pltpu.SMEM),
            pl.BlockSpec(memory_space=pltpu.SMEM),
            pl.BlockSpec(memory_space=pltpu.SMEM),
            pl.BlockSpec((16, 400, 400), lambda: (0, 0, 0)),
            pl.BlockSpec((25, 25, HID), lambda: (0, 0, 0)),
            pl.BlockSpec((1, HID), lambda: (0, 0)),
            pl.BlockSpec((16, 1), lambda: (0, 0)),
        ],
        out_specs=pl.BlockSpec((2, 16, 128), lambda: (0, 0, 0)),
        out_shape=jax.ShapeDtypeStruct((2, 16, 128), jnp.float32),
    )(w2, beta2, w3, beta3, out1d, wp, fce_b, deg_e)


def _tc_fcm(m_feat, wm, bm, deg_m):
    def body(x_ref, w_ref, b_ref, d_ref, o_ref):
        z = jnp.dot(x_ref[...], w_ref[...],
                    preferred_element_type=jnp.float32) + b_ref[...]
        dinv = lax.rsqrt(jnp.maximum(d_ref[...], 1.0))
        o_ref[0] = z * dinv

    return pl.pallas_call(
        body,
        grid=(2, 16),
        in_specs=[
            pl.BlockSpec((624, HID), lambda h, r: (r, 0)),
            pl.BlockSpec((HID, 128), lambda h, r: (0, h)),
            pl.BlockSpec((1, 128), lambda h, r: (0, h)),
            pl.BlockSpec((624, 1), lambda h, r: (r, 0)),
        ],
        out_specs=pl.BlockSpec((1, 624, 128), lambda h, r: (h, r, 0)),
        out_shape=jax.ShapeDtypeStruct((2, N_M, 128), jnp.float32),
    )(m_feat, wm, bm, deg_m)


def _tc_layer(agg, deg_in, deg_out, wr, b):
    def body(a_ref, di_ref, do_ref, w_ref, b_ref, o_ref):
        din = lax.rsqrt(jnp.maximum(di_ref[...], 1.0))
        a = a_ref[...]
        z = (jnp.dot(a[0] * din, w_ref[0], preferred_element_type=jnp.float32)
             + jnp.dot(a[1] * din, w_ref[1], preferred_element_type=jnp.float32)
             + b_ref[...])
        dout = lax.rsqrt(jnp.maximum(do_ref[...], 1.0))
        o_ref[0] = jnp.maximum(z, 0.0) * dout

    return pl.pallas_call(
        body,
        grid=(2, 25),
        in_specs=[
            pl.BlockSpec((2, 400, 128), lambda h, r: (0, r, 0)),
            pl.BlockSpec((400, 1), lambda h, r: (r, 0)),
            pl.BlockSpec((400, 1), lambda h, r: (r, 0)),
            pl.BlockSpec((2, 128, 128), lambda h, r: (0, 0, h)),
            pl.BlockSpec((1, 128), lambda h, r: (0, h)),
        ],
        out_specs=pl.BlockSpec((1, 400, 128), lambda h, r: (h, r, 0)),
        out_shape=jax.ShapeDtypeStruct((2, N, 128), jnp.float32),
    )(agg, deg_in, deg_out, wr, b)


def _tc_final(agg, deg_in, w3r, b3):
    def body(a_ref, di_ref, w3_ref, b_ref, o_ref):
        din = lax.rsqrt(jnp.maximum(di_ref[...], 1.0))
        a = a_ref[...]
        o_ref[...] = (
            jnp.dot(a[0] * din, w3_ref[0], preferred_element_type=jnp.float32)
            + jnp.dot(a[1] * din, w3_ref[1], preferred_element_type=jnp.float32)
            + b_ref[...])

    return pl.pallas_call(
        body,
        grid=(25,),
        in_specs=[
            pl.BlockSpec((2, 400, 128), lambda r: (0, r, 0)),
            pl.BlockSpec((400, 1), lambda r: (r, 0)),
            pl.BlockSpec((2, 128, 16), lambda r: (0, 0, 0)),
            pl.BlockSpec((1, 16), lambda r: (0, 0)),
        ],
        out_specs=pl.BlockSpec((400, 16), lambda r: (r, 0)),
        out_shape=jax.ShapeDtypeStruct((N, 16), jnp.float32),
    )(agg, deg_in, w3r, b3)


def kernel(e_feat, m_feat, edge_index, params):
    p = params
    gains = [p['bn_gamma'][i][0] / jnp.sqrt(jnp.float32(1.0 + 1e-5))
             for i in range(3)]
    w1 = p['conv_w'][0][0, 0] * gains[0]
    w2 = p['conv_w'][1][0, 0] * gains[1]
    w3 = p['conv_w'][2][0, 0] * gains[2]
    b1 = p['bn_beta'][0].reshape(1, 1)
    b2 = p['bn_beta'][1].reshape(1, 1)
    b3 = p['bn_beta'][2].reshape(1, 1)

    edge3 = edge_index.reshape(2, NCHUNK, CHUNK)

    deg = _sc_degrees(edge3)
    deg_out = deg[0].reshape(NPAD, 1)
    deg_in = deg[1].reshape(NPAD, 1)

    m1 = _conv1_matrix(w1)
    out1p = _tc_conv1(e_feat, m1, b1)
    out1d = (out1p.reshape(16, 400, 16, 25)
             .transpose(2, 1, 0, 3).reshape(16, 400, 400))
    wp = p['fc_e_W'].reshape(25, 25, HID).transpose(1, 0, 2)
    he2 = _tc_head(out1d, w2, b2, w3, b3, wp,
                   p['fc_e_b'].reshape(1, HID), deg_out[0:N_E])
    hm2 = _tc_fcm(m_feat, p['fc_m_W'], p['fc_m_b'].reshape(1, HID),
                  deg_out[N_E:N])
    hn0 = jnp.concatenate([he2, hm2], axis=1)

    z128 = jnp.zeros((NPAD, 128), jnp.float32)

    agg1 = _sc_aggregate(hn0.reshape(2 * N, 128), edge3, z128, 128)
    hn1 = _tc_layer(agg1, deg_in, deg_out,
                    p['gc_W'][0].reshape(2, 128, HID), p['gc_b'][0].reshape(1, HID))
    agg2 = _sc_aggregate(hn1.reshape(2 * N, 128), edge3, z128, 128)
    hn2 = _tc_layer(agg2, deg_in, deg_out,
                    p['gc_W'][1].reshape(2, 128, HID), p['gc_b'][1].reshape(1, HID))
    agg3 = _sc_aggregate(hn2.reshape(2 * N, 128), edge3, z128, 128)
    return _tc_final(agg3, deg_in, p['gc_W'][2].reshape(2, 128, 16),
                     p['gc_b'][2].reshape(1, 16))


# speedup vs baseline: 8.8902x; 1.4823x over previous
import functools

import jax
import jax.numpy as jnp
from jax import lax
from jax.experimental import pallas as pl
from jax.experimental.pallas import tpu as pltpu
import jax.experimental.pallas.tpu_sc as plsc

N_E = 16
N_M = 9984
N = 10000
NPAD = 10240
E = 160000
HID = 256
NC, NS = 2, 16
CHUNK = 128
NCHUNK = E // CHUNK
TPT = (NCHUNK + NS - 1) // NS


def _sc_degrees(edge3):
    mesh = plsc.VectorSubcoreMesh(core_axis_name="c", subcore_axis_name="s")

    @functools.partial(
        pl.kernel,
        out_type=jax.ShapeDtypeStruct((NC, NPAD), jnp.float32),
        mesh=mesh,
        scratch_types=[
            pltpu.VMEM_SHARED((NPAD,), jnp.float32),
            pltpu.VMEM((CHUNK,), jnp.int32),
            pltpu.VMEM((CHUNK,), jnp.float32),
            pltpu.VMEM((640,), jnp.float32),
        ],
    )
    def k(edge_hbm, out_hbm, hist_sh, idx_v, ones_v, z_v):
        c = lax.axis_index("c")
        s = lax.axis_index("s")
        for i in range(CHUNK // 16):
            ones_v[pl.ds(i * 16, 16)] = jnp.full((16,), 1.0, jnp.float32)
        for i in range(640 // 16):
            z_v[pl.ds(i * 16, 16)] = jnp.zeros((16,), jnp.float32)
        pltpu.sync_copy(z_v, hist_sh.at[pl.ds(s * 640, 640)])
        plsc.subcore_barrier()

        def body(t, carry):
            cj = s + NS * t

            @pl.when(cj < NCHUNK)
            def _():
                pltpu.sync_copy(edge_hbm.at[c, cj], idx_v)
                pltpu.sync_copy(ones_v, hist_sh.at[idx_v], add=True)

            return carry

        lax.fori_loop(0, TPT, body, 0)
        plsc.subcore_barrier()
        pltpu.sync_copy(hist_sh.at[pl.ds(s * 640, 640)],
                        out_hbm.at[c, pl.ds(s * 640, 640)])

    return k(edge3)


def _sc_aggregate(table, edge3, zeros, fh):
    mesh = plsc.VectorSubcoreMesh(core_axis_name="c", subcore_axis_name="s")

    NCB = 80

    @functools.partial(
        pl.kernel,
        out_type=jax.ShapeDtypeStruct((NC, NPAD, fh), jnp.float32),
        mesh=mesh,
        scratch_types=[
            pltpu.VMEM_SHARED((NPAD, fh), jnp.float32),
            pltpu.VMEM((NCB // 2, CHUNK), jnp.int32),
            pltpu.VMEM((NCB // 2, CHUNK), jnp.int32),
            pltpu.VMEM((CHUNK, fh), jnp.float32),
            pltpu.VMEM((CHUNK, fh), jnp.float32),
            pltpu.SemaphoreType.DMA,
            pltpu.SemaphoreType.DMA,
        ],
    )
    def k(table_hbm, edge_hbm, zero_hbm, out_hbm,
          acc_sh, src_v, dst_v, rows0, rows1, sem0, sem1):
        c = lax.axis_index("c")
        s = lax.axis_index("s")
        pltpu.sync_copy(zero_hbm.at[pl.ds(s * 640, 640)],
                        acc_sh.at[pl.ds(s * 640, 640)])
        cstart = s * NCB
        ntot = jnp.where(s == NS - 1, NCHUNK - (NS - 1) * NCB, NCB)
        off = c * N
        HB = NCB // 2
        plsc.subcore_barrier()

        def gather(t, buf, sem):
            pltpu.async_copy(table_hbm.at[src_v.at[t]], buf, sem)

        def gwait(t, buf, sem):
            pltpu.make_async_copy(table_hbm.at[src_v.at[t]], buf, sem).wait()

        def scatter(t, buf):
            pltpu.sync_copy(buf, acc_sh.at[dst_v.at[t]], add=True)

        for ph in range(2):
            n = jnp.clip(ntot - ph * HB, 0, HB)

            @pl.when(n > 0)
            def _():
                pltpu.sync_copy(edge_hbm.at[0, pl.ds(cstart + ph * HB, HB)],
                                src_v)
                pltpu.sync_copy(edge_hbm.at[1, pl.ds(cstart + ph * HB, HB)],
                                dst_v)

                def add_off(t, carry):
                    for i in range(CHUNK // 16):
                        src_v[t, pl.ds(i * 16, 16)] = (
                            src_v[t, pl.ds(i * 16, 16)] + off)
                    return carry

                lax.fori_loop(0, n, add_off, 0)
                gather(0, rows0, sem0)

                @pl.when(n > 1)
                def _():
                    gather(1, rows1, sem1)

                def body(i, carry):
                    t0 = 2 * i
                    t1 = 2 * i + 1

                    @pl.when(t0 < n)
                    def _():
                        gwait(t0, rows0, sem0)
                        scatter(t0, rows0)

                        @pl.when(t0 + 2 < n)
                        def _():
                            gather(t0 + 2, rows0, sem0)

                    @pl.when(t1 < n)
                    def _():
                        gwait(t1, rows1, sem1)
                        scatter(t1, rows1)

                        @pl.when(t1 + 2 < n)
                        def _():
                            gather(t1 + 2, rows1, sem1)

                    return carry

                lax.fori_loop(0, HB // 2, body, 0)

        plsc.subcore_barrier()
        pltpu.sync_copy(acc_sh.at[pl.ds(s * 640, 640)],
                        out_hbm.at[c, pl.ds(s * 640, 640)])

    return k(table, edge3, zeros)


def _lrelu(x):
    return jnp.where(x >= 0, x, 0.1 * x)


import numpy as _np

_JMAP = (16 * (_np.arange(400) % 25) + _np.arange(400) // 25)
_C1MASK = (_np.arange(1600)[:, None] // 4 == _JMAP[None, :]).astype(_np.float32)
_C1BSEL = _np.arange(1600) % 4


def _conv1_matrix(w1):
    cols = [ _C1MASK * w1[a][_C1BSEL][:, None] for a in range(4) ]
    return jnp.concatenate(cols, axis=1).astype(jnp.bfloat16)


def _tc_conv1(e_feat, m1, beta1):
    def body(b_ref, x_ref, m_ref, o_ref):
        xb = x_ref[0].astype(jnp.bfloat16)
        v = jnp.dot(xb, m_ref[...], preferred_element_type=jnp.float32)
        vr = v.reshape(400, 4, 1600)
        acc = (vr[:, 0, 0:400] + vr[:, 1, 400:800]
               + vr[:, 2, 800:1200] + vr[:, 3, 1200:1600])
        o_ref[0] = _lrelu(acc + b_ref[0, 0])

    return pl.pallas_call(
        body,
        grid=(16,),
        in_specs=[
            pl.BlockSpec(memory_space=pltpu.SMEM),
            pl.BlockSpec((1, 1600, 1600), lambda img: (img, 0, 0)),
            pl.BlockSpec((1600, 1600), lambda img: (0, 0)),
        ],
        out_specs=pl.BlockSpec((1, 400, 400), lambda img: (img, 0, 0)),
        out_shape=jax.ShapeDtypeStruct((16, 400, 400), jnp.float32),
    )(beta1, e_feat, m1)


def _tc_head(out1d, w2, beta2, w3, beta3, wp, fce_b, deg_e):
    def body(w2_ref, b2_ref, w3_ref, b3_ref, x_ref, wp_ref, fb_ref, de_ref,
             o_ref):
        out2 = []
        for u2 in range(4):
            acc = jnp.zeros((100, 400), jnp.float32)
            for a in range(4):
                for b in range(4):
                    xs = x_ref[4 * u2 + b].reshape(100, 4, 400)[:, a, :]
                    acc = acc + xs * w2_ref[a, b]
            out2.append(_lrelu(acc + b2_ref[0, 0]))
        acc3 = jnp.zeros((25, 400), jnp.float32)
        for a in range(4):
            for b in range(4):
                xs = out2[b].reshape(25, 4, 400)[:, a, :]
                acc3 = acc3 + xs * w3_ref[a, b]
        out3 = _lrelu(acc3 + b3_ref[0, 0])
        o3r = out3.T.reshape(16, 25, 25)
        he = jnp.zeros((16, HID), jnp.float32)
        for t3 in range(25):
            he = he + jnp.dot(o3r[:, t3, :], wp_ref[t3],
                              preferred_element_type=jnp.float32)
        dinv = lax.rsqrt(jnp.maximum(de_ref[...], 1.0))
        he = (he + fb_ref[...]) * dinv
        o_ref[0] = he[:, 0:128]
        o_ref[1] = he[:, 128:256]

    return pl.pallas_call(
        body,
        in_specs=[
            pl.BlockSpec(memory_space=pltpu.SMEM),
            pl.BlockSpec(memory_space=pltpu.SMEM),
            pl.BlockSpec(memory_space=pltpu.SMEM),
            pl.BlockSpec(memory_space=pltpu.SMEM),
            pl.BlockSpec((16, 400, 400), lambda: (0, 0, 0)),
            pl.BlockSpec((25, 25, HID), lambda: (0, 0, 0)),
            pl.BlockSpec((1, HID), lambda: (0, 0)),
            pl.BlockSpec((16, 1), lambda: (0, 0)),
        ],
        out_specs=pl.BlockSpec((2, 16, 128), lambda: (0, 0, 0)),
        out_shape=jax.ShapeDtypeStruct((2, 16, 128), jnp.float32),
    )(w2, beta2, w3, beta3, out1d, wp, fce_b, deg_e)


def _tc_fcm(m_feat, wm, bm, deg_m):
    def body(x_ref, w_ref, b_ref, d_ref, o_ref):
        z = jnp.dot(x_ref[...], w_ref[...],
                    preferred_element_type=jnp.float32) + b_ref[...]
        dinv = lax.rsqrt(jnp.maximum(d_ref[...], 1.0))
        o_ref[0] = z * dinv

    return pl.pallas_call(
        body,
        grid=(2, 16),
        in_specs=[
            pl.BlockSpec((624, HID), lambda h, r: (r, 0)),
            pl.BlockSpec((HID, 128), lambda h, r: (0, h)),
            pl.BlockSpec((1, 128), lambda h, r: (0, h)),
            pl.BlockSpec((624, 1), lambda h, r: (r, 0)),
        ],
        out_specs=pl.BlockSpec((1, 624, 128), lambda h, r: (h, r, 0)),
        out_shape=jax.ShapeDtypeStruct((2, N_M, 128), jnp.float32),
    )(m_feat, wm, bm, deg_m)


def _tc_layer(agg, deg_in, deg_out, wr, b):
    def body(a_ref, di_ref, do_ref, w_ref, b_ref, o_ref):
        din = lax.rsqrt(jnp.maximum(di_ref[...], 1.0))
        a = a_ref[...]
        z = (jnp.dot(a[0] * din, w_ref[0], preferred_element_type=jnp.float32)
             + jnp.dot(a[1] * din, w_ref[1], preferred_element_type=jnp.float32)
             + b_ref[...])
        dout = lax.rsqrt(jnp.maximum(do_ref[...], 1.0))
        o_ref[0] = jnp.maximum(z, 0.0) * dout

    return pl.pallas_call(
        body,
        grid=(2, 25),
        in_specs=[
            pl.BlockSpec((2, 400, 128), lambda h, r: (0, r, 0)),
            pl.BlockSpec((400, 1), lambda h, r: (r, 0)),
            pl.BlockSpec((400, 1), lambda h, r: (r, 0)),
            pl.BlockSpec((2, 128, 128), lambda h, r: (0, 0, h)),
            pl.BlockSpec((1, 128), lambda h, r: (0, h)),
        ],
        out_specs=pl.BlockSpec((1, 400, 128), lambda h, r: (h, r, 0)),
        out_shape=jax.ShapeDtypeStruct((2, N, 128), jnp.float32),
    )(agg, deg_in, deg_out, wr, b)


def _tc_final(agg, deg_in, w3r, b3):
    def body(a_ref, di_ref, w3_ref, b_ref, o_ref):
        din = lax.rsqrt(jnp.maximum(di_ref[...], 1.0))
        a = a_ref[...]
        o_ref[...] = (
            jnp.dot(a[0] * din, w3_ref[0], preferred_element_type=jnp.float32)
            + jnp.dot(a[1] * din, w3_ref[1], preferred_element_type=jnp.float32)
            + b_ref[...])

    return pl.pallas_call(
        body,
        grid=(25,),
        in_specs=[
            pl.BlockSpec((2, 400, 128), lambda r: (0, r, 0)),
            pl.BlockSpec((400, 1), lambda r: (r, 0)),
            pl.BlockSpec((2, 128, 16), lambda r: (0, 0, 0)),
            pl.BlockSpec((1, 16), lambda r: (0, 0)),
        ],
        out_specs=pl.BlockSpec((400, 16), lambda r: (r, 0)),
        out_shape=jax.ShapeDtypeStruct((N, 16), jnp.float32),
    )(agg, deg_in, w3r, b3)


def kernel(e_feat, m_feat, edge_index, params):
    p = params
    gains = [p['bn_gamma'][i][0] / jnp.sqrt(jnp.float32(1.0 + 1e-5))
             for i in range(3)]
    w1 = p['conv_w'][0][0, 0] * gains[0]
    w2 = p['conv_w'][1][0, 0] * gains[1]
    w3 = p['conv_w'][2][0, 0] * gains[2]
    b1 = p['bn_beta'][0].reshape(1, 1)
    b2 = p['bn_beta'][1].reshape(1, 1)
    b3 = p['bn_beta'][2].reshape(1, 1)

    edge3 = edge_index.reshape(2, NCHUNK, CHUNK)
    edge3p = jnp.pad(edge3, ((0, 0), (0, NS * 80 - NCHUNK), (0, 0)))

    deg = _sc_degrees(edge3)
    deg_out = deg[0].reshape(NPAD, 1)
    deg_in = deg[1].reshape(NPAD, 1)

    m1 = _conv1_matrix(w1)
    out1p = _tc_conv1(e_feat, m1, b1)
    out1d = (out1p.reshape(16, 400, 16, 25)
             .transpose(2, 1, 0, 3).reshape(16, 400, 400))
    wp = p['fc_e_W'].reshape(25, 25, HID).transpose(1, 0, 2)
    he2 = _tc_head(out1d, w2, b2, w3, b3, wp,
                   p['fc_e_b'].reshape(1, HID), deg_out[0:N_E])
    hm2 = _tc_fcm(m_feat, p['fc_m_W'], p['fc_m_b'].reshape(1, HID),
                  deg_out[N_E:N])
    hn0 = jnp.concatenate([he2, hm2], axis=1)

    z128 = jnp.zeros((NPAD, 128), jnp.float32)

    agg1 = _sc_aggregate(hn0.reshape(2 * N, 128), edge3p, z128, 128)
    hn1 = _tc_layer(agg1, deg_in, deg_out,
                    p['gc_W'][0].reshape(2, 128, HID), p['gc_b'][0].reshape(1, HID))
    agg2 = _sc_aggregate(hn1.reshape(2 * N, 128), edge3p, z128, 128)
    hn2 = _tc_layer(agg2, deg_in, deg_out,
                    p['gc_W'][1].reshape(2, 128, HID), p['gc_b'][1].reshape(1, HID))
    agg3 = _sc_aggregate(hn2.reshape(2 * N, 128), edge3p, z128, 128)
    return _tc_final(agg3, deg_in, p['gc_W'][2].reshape(2, 128, 16),
                     p['gc_b'][2].reshape(1, 16))


# speedup vs baseline: 10.9181x; 1.2281x over previous
import functools

import jax
import jax.numpy as jnp
from jax import lax
from jax.experimental import pallas as pl
from jax.experimental.pallas import tpu as pltpu
import jax.experimental.pallas.tpu_sc as plsc

N_E = 16
N_M = 9984
N = 10000
NPAD = 10240
E = 160000
HID = 256
NC, NS = 2, 16
CHUNK = 128
NCHUNK = E // CHUNK
TPT = (NCHUNK + NS - 1) // NS


def _sc_degrees(edge3):
    mesh = plsc.VectorSubcoreMesh(core_axis_name="c", subcore_axis_name="s")

    @functools.partial(
        pl.kernel,
        out_type=jax.ShapeDtypeStruct((NC, NPAD), jnp.float32),
        mesh=mesh,
        scratch_types=[
            pltpu.VMEM_SHARED((NPAD,), jnp.float32),
            pltpu.VMEM((CHUNK,), jnp.int32),
            pltpu.VMEM((CHUNK,), jnp.float32),
            pltpu.VMEM((640,), jnp.float32),
        ],
    )
    def k(edge_hbm, out_hbm, hist_sh, idx_v, ones_v, z_v):
        c = lax.axis_index("c")
        s = lax.axis_index("s")
        for i in range(CHUNK // 16):
            ones_v[pl.ds(i * 16, 16)] = jnp.full((16,), 1.0, jnp.float32)
        for i in range(640 // 16):
            z_v[pl.ds(i * 16, 16)] = jnp.zeros((16,), jnp.float32)
        pltpu.sync_copy(z_v, hist_sh.at[pl.ds(s * 640, 640)])
        plsc.subcore_barrier()

        def body(t, carry):
            cj = s + NS * t

            @pl.when(cj < NCHUNK)
            def _():
                pltpu.sync_copy(edge_hbm.at[c, cj], idx_v)
                pltpu.sync_copy(ones_v, hist_sh.at[idx_v], add=True)

            return carry

        lax.fori_loop(0, TPT, body, 0)
        plsc.subcore_barrier()
        pltpu.sync_copy(hist_sh.at[pl.ds(s * 640, 640)],
                        out_hbm.at[c, pl.ds(s * 640, 640)])

    return k(edge3)


def _sc_aggregate(table, edge3, zeros, fh):
    mesh = plsc.VectorSubcoreMesh(core_axis_name="c", subcore_axis_name="s")

    NCB = 80

    @functools.partial(
        pl.kernel,
        out_type=jax.ShapeDtypeStruct((NC, NPAD, fh), jnp.float32),
        mesh=mesh,
        scratch_types=[
            pltpu.VMEM_SHARED((NPAD, fh), jnp.float32),
            pltpu.VMEM((NCB // 2, CHUNK), jnp.int32),
            pltpu.VMEM((NCB // 2, CHUNK), jnp.int32),
            pltpu.VMEM((CHUNK, fh), jnp.float32),
            pltpu.VMEM((CHUNK, fh), jnp.float32),
            pltpu.SemaphoreType.DMA,
            pltpu.SemaphoreType.DMA,
        ],
    )
    def k(table_hbm, edge_hbm, zero_hbm, out_hbm,
          acc_sh, src_v, dst_v, rows0, rows1, sem0, sem1):
        c = lax.axis_index("c")
        s = lax.axis_index("s")
        pltpu.sync_copy(zero_hbm.at[pl.ds(s * 640, 640)],
                        acc_sh.at[pl.ds(s * 640, 640)])
        cstart = s * NCB
        ntot = jnp.where(s == NS - 1, NCHUNK - (NS - 1) * NCB, NCB)
        off = c * N
        HB = NCB // 2
        plsc.subcore_barrier()

        def gather(t, buf, sem):
            pltpu.async_copy(table_hbm.at[src_v.at[t]], buf, sem)

        def gwait(t, buf, sem):
            pltpu.make_async_copy(table_hbm.at[src_v.at[t]], buf, sem).wait()

        def scatter(t, buf):
            pltpu.sync_copy(buf, acc_sh.at[dst_v.at[t]], add=True)

        for ph in range(2):
            n = jnp.clip(ntot - ph * HB, 0, HB)

            @pl.when(n > 0)
            def _():
                pltpu.sync_copy(edge_hbm.at[0, pl.ds(cstart + ph * HB, HB)],
                                src_v)
                pltpu.sync_copy(edge_hbm.at[1, pl.ds(cstart + ph * HB, HB)],
                                dst_v)

                def add_off(t, carry):
                    for i in range(CHUNK // 16):
                        src_v[t, pl.ds(i * 16, 16)] = (
                            src_v[t, pl.ds(i * 16, 16)] + off)
                    return carry

                lax.fori_loop(0, n, add_off, 0)
                gather(0, rows0, sem0)

                @pl.when(n > 1)
                def _():
                    gather(1, rows1, sem1)

                def body(i, carry):
                    t0 = 2 * i
                    t1 = 2 * i + 1

                    @pl.when(t0 < n)
                    def _():
                        gwait(t0, rows0, sem0)
                        scatter(t0, rows0)

                        @pl.when(t0 + 2 < n)
                        def _():
                            gather(t0 + 2, rows0, sem0)

                    @pl.when(t1 < n)
                    def _():
                        gwait(t1, rows1, sem1)
                        scatter(t1, rows1)

                        @pl.when(t1 + 2 < n)
                        def _():
                            gather(t1 + 2, rows1, sem1)

                    return carry

                lax.fori_loop(0, HB // 2, body, 0)

        plsc.subcore_barrier()
        pltpu.sync_copy(acc_sh.at[pl.ds(s * 640, 640)],
                        out_hbm.at[c, pl.ds(s * 640, 640)])

    return k(table, edge3, zeros)


def _lrelu(x):
    return jnp.where(x >= 0, x, 0.1 * x)


import numpy as _np

_C1MASK = (_np.arange(512)[:, None] // 4 ==
           _np.arange(128)[None, :]).astype(_np.float32)
_C1BSEL = _np.arange(512) % 4


def _conv1_matrix(w1):
    return jnp.stack([_C1MASK * w1[a][_C1BSEL][:, None]
                      for a in range(4)]).astype(jnp.bfloat16)


def _tc_conv1(e_feat, m1, beta1):
    def body(b_ref, x_ref, m_ref, o_ref):
        xb = x_ref[0].astype(jnp.bfloat16)
        m = m_ref[...]
        blocks = []
        for jb in range(4):
            kw = 512 if jb < 3 else 64
            nw = 128 if jb < 3 else 16
            xs = xb[:, 512 * jb:512 * jb + kw]
            acc = jnp.zeros((400, nw), jnp.float32)
            for a in range(4):
                t = jnp.dot(xs, m[a, 0:kw, 0:nw],
                            preferred_element_type=jnp.float32)
                acc = acc + t.reshape(400, 4, nw)[:, a, :]
            blocks.append(acc)
        res = jnp.concatenate(blocks, axis=1)
        o_ref[0] = _lrelu(res + b_ref[0, 0])

    return pl.pallas_call(
        body,
        grid=(16,),
        in_specs=[
            pl.BlockSpec(memory_space=pltpu.SMEM),
            pl.BlockSpec((1, 1600, 1600), lambda img: (img, 0, 0)),
            pl.BlockSpec((4, 512, 128), lambda img: (0, 0, 0)),
        ],
        out_specs=pl.BlockSpec((1, 400, 400), lambda img: (img, 0, 0)),
        out_shape=jax.ShapeDtypeStruct((16, 400, 400), jnp.float32),
    )(beta1, e_feat, m1)


def _tc_head(out1d, w2, beta2, w3, beta3, wp, fce_b, deg_e):
    def body(w2_ref, b2_ref, w3_ref, b3_ref, x_ref, wp_ref, fb_ref, de_ref,
             o_ref):
        out2 = []
        for u2 in range(4):
            acc = jnp.zeros((100, 400), jnp.float32)
            for a in range(4):
                for b in range(4):
                    xs = x_ref[4 * u2 + b].reshape(100, 4, 400)[:, a, :]
                    acc = acc + xs * w2_ref[a, b]
            out2.append(_lrelu(acc + b2_ref[0, 0]))
        acc3 = jnp.zeros((25, 400), jnp.float32)
        for a in range(4):
            for b in range(4):
                xs = out2[b].reshape(25, 4, 400)[:, a, :]
                acc3 = acc3 + xs * w3_ref[a, b]
        out3 = _lrelu(acc3 + b3_ref[0, 0])
        o3r = out3.T.reshape(16, 25, 25)
        he = jnp.zeros((16, HID), jnp.float32)
        for t3 in range(25):
            he = he + jnp.dot(o3r[:, t3, :], wp_ref[t3],
                              preferred_element_type=jnp.float32)
        dinv = lax.rsqrt(jnp.maximum(de_ref[...], 1.0))
        he = (he + fb_ref[...]) * dinv
        o_ref[0] = he[:, 0:128]
        o_ref[1] = he[:, 128:256]

    return pl.pallas_call(
        body,
        in_specs=[
            pl.BlockSpec(memory_space=pltpu.SMEM),
            pl.BlockSpec(memory_space=pltpu.SMEM),
            pl.BlockSpec(memory_space=pltpu.SMEM),
            pl.BlockSpec(memory_space=pltpu.SMEM),
            pl.BlockSpec((16, 400, 400), lambda: (0, 0, 0)),
            pl.BlockSpec((25, 25, HID), lambda: (0, 0, 0)),
            pl.BlockSpec((1, HID), lambda: (0, 0)),
            pl.BlockSpec((16, 1), lambda: (0, 0)),
        ],
        out_specs=pl.BlockSpec((2, 16, 128), lambda: (0, 0, 0)),
        out_shape=jax.ShapeDtypeStruct((2, 16, 128), jnp.float32),
    )(w2, beta2, w3, beta3, out1d, wp, fce_b, deg_e)


def _tc_fcm(m_feat, wm, bm, deg_m):
    def body(x_ref, w_ref, b_ref, d_ref, o_ref):
        z = jnp.dot(x_ref[...], w_ref[...],
                    preferred_element_type=jnp.float32) + b_ref[...]
        dinv = lax.rsqrt(jnp.maximum(d_ref[...], 1.0))
        o_ref[0] = z * dinv

    return pl.pallas_call(
        body,
        grid=(2, 16),
        in_specs=[
            pl.BlockSpec((624, HID), lambda h, r: (r, 0)),
            pl.BlockSpec((HID, 128), lambda h, r: (0, h)),
            pl.BlockSpec((1, 128), lambda h, r: (0, h)),
            pl.BlockSpec((624, 1), lambda h, r: (r, 0)),
        ],
        out_specs=pl.BlockSpec((1, 624, 128), lambda h, r: (h, r, 0)),
        out_shape=jax.ShapeDtypeStruct((2, N_M, 128), jnp.float32),
    )(m_feat, wm, bm, deg_m)


def _tc_layer(agg, deg_in, deg_out, wr, b):
    def body(a_ref, di_ref, do_ref, w_ref, b_ref, o_ref):
        din = lax.rsqrt(jnp.maximum(di_ref[...], 1.0))
        a = a_ref[...]
        z = (jnp.dot(a[0] * din, w_ref[0], preferred_element_type=jnp.float32)
             + jnp.dot(a[1] * din, w_ref[1], preferred_element_type=jnp.float32)
             + b_ref[...])
        dout = lax.rsqrt(jnp.maximum(do_ref[...], 1.0))
        o_ref[0] = jnp.maximum(z, 0.0) * dout

    return pl.pallas_call(
        body,
        grid=(2, 25),
        in_specs=[
            pl.BlockSpec((2, 400, 128), lambda h, r: (0, r, 0)),
            pl.BlockSpec((400, 1), lambda h, r: (r, 0)),
            pl.BlockSpec((400, 1), lambda h, r: (r, 0)),
            pl.BlockSpec((2, 128, 128), lambda h, r: (0, 0, h)),
            pl.BlockSpec((1, 128), lambda h, r: (0, h)),
        ],
        out_specs=pl.BlockSpec((1, 400, 128), lambda h, r: (h, r, 0)),
        out_shape=jax.ShapeDtypeStruct((2, N, 128), jnp.float32),
    )(agg, deg_in, deg_out, wr, b)


def _tc_final(agg, deg_in, w3r, b3):
    def body(a_ref, di_ref, w3_ref, b_ref, o_ref):
        din = lax.rsqrt(jnp.maximum(di_ref[...], 1.0))
        a = a_ref[...]
        o_ref[...] = (
            jnp.dot(a[0] * din, w3_ref[0], preferred_element_type=jnp.float32)
            + jnp.dot(a[1] * din, w3_ref[1], preferred_element_type=jnp.float32)
            + b_ref[...])

    return pl.pallas_call(
        body,
        grid=(25,),
        in_specs=[
            pl.BlockSpec((2, 400, 128), lambda r: (0, r, 0)),
            pl.BlockSpec((400, 1), lambda r: (r, 0)),
            pl.BlockSpec((2, 128, 16), lambda r: (0, 0, 0)),
            pl.BlockSpec((1, 16), lambda r: (0, 0)),
        ],
        out_specs=pl.BlockSpec((400, 16), lambda r: (r, 0)),
        out_shape=jax.ShapeDtypeStruct((N, 16), jnp.float32),
    )(agg, deg_in, w3r, b3)


def kernel(e_feat, m_feat, edge_index, params):
    p = params
    gains = [p['bn_gamma'][i][0] / jnp.sqrt(jnp.float32(1.0 + 1e-5))
             for i in range(3)]
    w1 = p['conv_w'][0][0, 0] * gains[0]
    w2 = p['conv_w'][1][0, 0] * gains[1]
    w3 = p['conv_w'][2][0, 0] * gains[2]
    b1 = p['bn_beta'][0].reshape(1, 1)
    b2 = p['bn_beta'][1].reshape(1, 1)
    b3 = p['bn_beta'][2].reshape(1, 1)

    edge3 = edge_index.reshape(2, NCHUNK, CHUNK)
    edge3p = jnp.pad(edge3, ((0, 0), (0, NS * 80 - NCHUNK), (0, 0)))

    deg = _sc_degrees(edge3)
    deg_out = deg[0].reshape(NPAD, 1)
    deg_in = deg[1].reshape(NPAD, 1)

    m1 = _conv1_matrix(w1)
    out1p = _tc_conv1(e_feat, m1, b1)
    out1d = (out1p.reshape(16, 400, 25, 16)
             .transpose(3, 1, 0, 2).reshape(16, 400, 400))
    wp = p['fc_e_W'].reshape(25, 25, HID).transpose(1, 0, 2)
    he2 = _tc_head(out1d, w2, b2, w3, b3, wp,
                   p['fc_e_b'].reshape(1, HID), deg_out[0:N_E])
    hm2 = _tc_fcm(m_feat, p['fc_m_W'], p['fc_m_b'].reshape(1, HID),
                  deg_out[N_E:N])
    hn0 = jnp.concatenate([he2, hm2], axis=1)

    z128 = jnp.zeros((NPAD, 128), jnp.float32)

    agg1 = _sc_aggregate(hn0.reshape(2 * N, 128), edge3p, z128, 128)
    hn1 = _tc_layer(agg1, deg_in, deg_out,
                    p['gc_W'][0].reshape(2, 128, HID), p['gc_b'][0].reshape(1, HID))
    agg2 = _sc_aggregate(hn1.reshape(2 * N, 128), edge3p, z128, 128)
    hn2 = _tc_layer(agg2, deg_in, deg_out,
                    p['gc_W'][1].reshape(2, 128, HID), p['gc_b'][1].reshape(1, HID))
    agg3 = _sc_aggregate(hn2.reshape(2 * N, 128), edge3p, z128, 128)
    return _tc_final(agg3, deg_in, p['gc_W'][2].reshape(2, 128, 16),
                     p['gc_b'][2].reshape(1, 16))


# speedup vs baseline: 10.9390x; 1.0019x over previous
import functools

import jax
import jax.numpy as jnp
from jax import lax
from jax.experimental import pallas as pl
from jax.experimental.pallas import tpu as pltpu
import jax.experimental.pallas.tpu_sc as plsc

N_E = 16
N_M = 9984
N = 10000
NPAD = 10240
E = 160000
HID = 256
NC, NS = 2, 16
CHUNK = 128
NCHUNK = E // CHUNK
TPT = (NCHUNK + NS - 1) // NS


def _sc_degrees(edge3p):
    mesh = plsc.VectorSubcoreMesh(core_axis_name="c", subcore_axis_name="s")
    NCB = 80

    @functools.partial(
        pl.kernel,
        out_type=jax.ShapeDtypeStruct((NC, NPAD), jnp.float32),
        mesh=mesh,
        scratch_types=[
            pltpu.VMEM_SHARED((NPAD,), jnp.float32),
            pltpu.VMEM((NCB, CHUNK), jnp.int32),
            pltpu.VMEM((CHUNK,), jnp.float32),
            pltpu.VMEM((640,), jnp.float32),
            pltpu.SemaphoreType.DMA,
        ],
    )
    def k(edge_hbm, out_hbm, hist_sh, idx_v, ones_v, z_v, sem):
        c = lax.axis_index("c")
        s = lax.axis_index("s")
        for i in range(CHUNK // 16):
            ones_v[pl.ds(i * 16, 16)] = jnp.full((16,), 1.0, jnp.float32)
        for i in range(640 // 16):
            z_v[pl.ds(i * 16, 16)] = jnp.zeros((16,), jnp.float32)
        pltpu.sync_copy(z_v, hist_sh.at[pl.ds(s * 640, 640)])
        cstart = s * NCB
        n = jnp.where(s == NS - 1, NCHUNK - (NS - 1) * NCB, NCB)
        pltpu.sync_copy(edge_hbm.at[c, pl.ds(cstart, NCB)], idx_v)
        plsc.subcore_barrier()

        def fire(t, carry):
            @pl.when(t < n)
            def _():
                pltpu.async_copy(ones_v, hist_sh.at[idx_v.at[t]], sem,
                                 add=True)
            return carry

        def drain(t, carry):
            @pl.when(t < n)
            def _():
                pltpu.make_async_copy(ones_v, hist_sh.at[idx_v.at[t]],
                                      sem).wait()
            return carry

        lax.fori_loop(0, NCB, fire, 0)
        lax.fori_loop(0, NCB, drain, 0)
        plsc.subcore_barrier()
        pltpu.sync_copy(hist_sh.at[pl.ds(s * 640, 640)],
                        out_hbm.at[c, pl.ds(s * 640, 640)])

    return k(edge3p)


def _sc_aggregate(table, edge3, zeros, fh):
    mesh = plsc.VectorSubcoreMesh(core_axis_name="c", subcore_axis_name="s")

    NCB = 80

    @functools.partial(
        pl.kernel,
        out_type=jax.ShapeDtypeStruct((NC, NPAD, fh), jnp.float32),
        mesh=mesh,
        scratch_types=[
            pltpu.VMEM_SHARED((NPAD, fh), jnp.float32),
            pltpu.VMEM((NCB // 2, CHUNK), jnp.int32),
            pltpu.VMEM((NCB // 2, CHUNK), jnp.int32),
            pltpu.VMEM((CHUNK, fh), jnp.float32),
            pltpu.VMEM((CHUNK, fh), jnp.float32),
            pltpu.SemaphoreType.DMA,
            pltpu.SemaphoreType.DMA,
        ],
    )
    def k(table_hbm, edge_hbm, zero_hbm, out_hbm,
          acc_sh, src_v, dst_v, rows0, rows1, sem0, sem1):
        c = lax.axis_index("c")
        s = lax.axis_index("s")
        pltpu.sync_copy(zero_hbm.at[pl.ds(s * 640, 640)],
                        acc_sh.at[pl.ds(s * 640, 640)])
        cstart = s * NCB
        ntot = jnp.where(s == NS - 1, NCHUNK - (NS - 1) * NCB, NCB)
        off = c * N
        HB = NCB // 2
        plsc.subcore_barrier()

        def gather(t, buf, sem):
            pltpu.async_copy(table_hbm.at[src_v.at[t]], buf, sem)

        def gwait(t, buf, sem):
            pltpu.make_async_copy(table_hbm.at[src_v.at[t]], buf, sem).wait()

        def scatter(t, buf):
            pltpu.sync_copy(buf, acc_sh.at[dst_v.at[t]], add=True)

        for ph in range(2):
            n = jnp.clip(ntot - ph * HB, 0, HB)

            @pl.when(n > 0)
            def _():
                pltpu.sync_copy(edge_hbm.at[0, pl.ds(cstart + ph * HB, HB)],
                                src_v)
                pltpu.sync_copy(edge_hbm.at[1, pl.ds(cstart + ph * HB, HB)],
                                dst_v)

                def add_off(t, carry):
                    for i in range(CHUNK // 16):
                        src_v[t, pl.ds(i * 16, 16)] = (
                            src_v[t, pl.ds(i * 16, 16)] + off)
                    return carry

                lax.fori_loop(0, n, add_off, 0)
                gather(0, rows0, sem0)

                @pl.when(n > 1)
                def _():
                    gather(1, rows1, sem1)

                def body(i, carry):
                    t0 = 2 * i
                    t1 = 2 * i + 1

                    @pl.when(t0 < n)
                    def _():
                        gwait(t0, rows0, sem0)
                        scatter(t0, rows0)

                        @pl.when(t0 + 2 < n)
                        def _():
                            gather(t0 + 2, rows0, sem0)

                    @pl.when(t1 < n)
                    def _():
                        gwait(t1, rows1, sem1)
                        scatter(t1, rows1)

                        @pl.when(t1 + 2 < n)
                        def _():
                            gather(t1 + 2, rows1, sem1)

                    return carry

                lax.fori_loop(0, HB // 2, body, 0)

        plsc.subcore_barrier()
        pltpu.sync_copy(acc_sh.at[pl.ds(s * 640, 640)],
                        out_hbm.at[c, pl.ds(s * 640, 640)])

    return k(table, edge3, zeros)


def _lrelu(x):
    return jnp.where(x >= 0, x, 0.1 * x)


import numpy as _np

_C1MASK = (_np.arange(512)[:, None] // 4 ==
           _np.arange(128)[None, :]).astype(_np.float32)
_C1BSEL = _np.arange(512) % 4


def _conv1_matrix(w1):
    return jnp.stack([_C1MASK * w1[a][_C1BSEL][:, None]
                      for a in range(4)]).astype(jnp.bfloat16)


def _tc_conv1(e_feat, m1, beta1):
    def body(b_ref, x_ref, m_ref, o_ref):
        xb = x_ref[0].astype(jnp.bfloat16)
        m = m_ref[...]
        blocks = []
        for jb in range(4):
            kw = 512 if jb < 3 else 64
            nw = 128 if jb < 3 else 16
            xs = xb[:, 512 * jb:512 * jb + kw]
            acc = jnp.zeros((400, nw), jnp.float32)
            for a in range(4):
                t = jnp.dot(xs, m[a, 0:kw, 0:nw],
                            preferred_element_type=jnp.float32)
                acc = acc + t.reshape(400, 4, nw)[:, a, :]
            blocks.append(acc)
        res = jnp.concatenate(blocks, axis=1)
        o_ref[0] = _lrelu(res + b_ref[0, 0])

    return pl.pallas_call(
        body,
        grid=(16,),
        in_specs=[
            pl.BlockSpec(memory_space=pltpu.SMEM),
            pl.BlockSpec((1, 1600, 1600), lambda img: (img, 0, 0)),
            pl.BlockSpec((4, 512, 128), lambda img: (0, 0, 0)),
        ],
        out_specs=pl.BlockSpec((1, 400, 400), lambda img: (img, 0, 0)),
        out_shape=jax.ShapeDtypeStruct((16, 400, 400), jnp.float32),
    )(beta1, e_feat, m1)


def _tc_head(out1d, w2, beta2, w3, beta3, wp, fce_b, deg_e):
    def body(w2_ref, b2_ref, w3_ref, b3_ref, x_ref, wp_ref, fb_ref, de_ref,
             o_ref):
        out2 = []
        for u2 in range(4):
            acc = jnp.zeros((100, 400), jnp.float32)
            for a in range(4):
                for b in range(4):
                    xs = x_ref[4 * u2 + b].reshape(100, 4, 400)[:, a, :]
                    acc = acc + xs * w2_ref[a, b]
            out2.append(_lrelu(acc + b2_ref[0, 0]))
        acc3 = jnp.zeros((25, 400), jnp.float32)
        for a in range(4):
            for b in range(4):
                xs = out2[b].reshape(25, 4, 400)[:, a, :]
                acc3 = acc3 + xs * w3_ref[a, b]
        out3 = _lrelu(acc3 + b3_ref[0, 0])
        o3r = out3.T.reshape(16, 25, 25)
        he = jnp.zeros((16, HID), jnp.float32)
        for t3 in range(25):
            he = he + jnp.dot(o3r[:, t3, :], wp_ref[t3],
                              preferred_element_type=jnp.float32)
        dinv = lax.rsqrt(jnp.maximum(de_ref[...], 1.0))
        he = (he + fb_ref[...]) * dinv
        o_ref[0] = he[:, 0:128]
        o_ref[1] = he[:, 128:256]

    return pl.pallas_call(
        body,
        in_specs=[
            pl.BlockSpec(memory_space=pltpu.SMEM),
            pl.BlockSpec(memory_space=pltpu.SMEM),
            pl.BlockSpec(memory_space=pltpu.SMEM),
            pl.BlockSpec(memory_space=pltpu.SMEM),
            pl.BlockSpec((16, 400, 400), lambda: (0, 0, 0)),
            pl.BlockSpec((25, 25, HID), lambda: (0, 0, 0)),
            pl.BlockSpec((1, HID), lambda: (0, 0)),
            pl.BlockSpec((16, 1), lambda: (0, 0)),
        ],
        out_specs=pl.BlockSpec((2, 16, 128), lambda: (0, 0, 0)),
        out_shape=jax.ShapeDtypeStruct((2, 16, 128), jnp.float32),
    )(w2, beta2, w3, beta3, out1d, wp, fce_b, deg_e)


def _tc_fcm(m_feat, wm, bm, deg_m):
    def body(x_ref, w_ref, b_ref, d_ref, o_ref):
        z = jnp.dot(x_ref[...], w_ref[...],
                    preferred_element_type=jnp.float32) + b_ref[...]
        dinv = lax.rsqrt(jnp.maximum(d_ref[...], 1.0))
        o_ref[0] = z * dinv

    return pl.pallas_call(
        body,
        grid=(2, 16),
        in_specs=[
            pl.BlockSpec((624, HID), lambda h, r: (r, 0)),
            pl.BlockSpec((HID, 128), lambda h, r: (0, h)),
            pl.BlockSpec((1, 128), lambda h, r: (0, h)),
            pl.BlockSpec((624, 1), lambda h, r: (r, 0)),
        ],
        out_specs=pl.BlockSpec((1, 624, 128), lambda h, r: (h, r, 0)),
        out_shape=jax.ShapeDtypeStruct((2, N_M, 128), jnp.float32),
    )(m_feat, wm, bm, deg_m)


def _tc_layer(agg, deg_in, deg_out, wr, b):
    def body(a_ref, di_ref, do_ref, w_ref, b_ref, o_ref):
        din = lax.rsqrt(jnp.maximum(di_ref[...], 1.0))
        a = a_ref[...]
        z = (jnp.dot(a[0] * din, w_ref[0], preferred_element_type=jnp.float32)
             + jnp.dot(a[1] * din, w_ref[1], preferred_element_type=jnp.float32)
             + b_ref[...])
        dout = lax.rsqrt(jnp.maximum(do_ref[...], 1.0))
        o_ref[0] = jnp.maximum(z, 0.0) * dout

    return pl.pallas_call(
        body,
        grid=(2, 25),
        in_specs=[
            pl.BlockSpec((2, 400, 128), lambda h, r: (0, r, 0)),
            pl.BlockSpec((400, 1), lambda h, r: (r, 0)),
            pl.BlockSpec((400, 1), lambda h, r: (r, 0)),
            pl.BlockSpec((2, 128, 128), lambda h, r: (0, 0, h)),
            pl.BlockSpec((1, 128), lambda h, r: (0, h)),
        ],
        out_specs=pl.BlockSpec((1, 400, 128), lambda h, r: (h, r, 0)),
        out_shape=jax.ShapeDtypeStruct((2, N, 128), jnp.float32),
    )(agg, deg_in, deg_out, wr, b)


def _tc_final(agg, deg_in, w3r, b3):
    def body(a_ref, di_ref, w3_ref, b_ref, o_ref):
        din = lax.rsqrt(jnp.maximum(di_ref[...], 1.0))
        a = a_ref[...]
        o_ref[...] = (
            jnp.dot(a[0] * din, w3_ref[0], preferred_element_type=jnp.float32)
            + jnp.dot(a[1] * din, w3_ref[1], preferred_element_type=jnp.float32)
            + b_ref[...])

    return pl.pallas_call(
        body,
        grid=(25,),
        in_specs=[
            pl.BlockSpec((2, 400, 128), lambda r: (0, r, 0)),
            pl.BlockSpec((400, 1), lambda r: (r, 0)),
            pl.BlockSpec((2, 128, 16), lambda r: (0, 0, 0)),
            pl.BlockSpec((1, 16), lambda r: (0, 0)),
        ],
        out_specs=pl.BlockSpec((400, 16), lambda r: (r, 0)),
        out_shape=jax.ShapeDtypeStruct((N, 16), jnp.float32),
    )(agg, deg_in, w3r, b3)


def kernel(e_feat, m_feat, edge_index, params):
    p = params
    gains = [p['bn_gamma'][i][0] / jnp.sqrt(jnp.float32(1.0 + 1e-5))
             for i in range(3)]
    w1 = p['conv_w'][0][0, 0] * gains[0]
    w2 = p['conv_w'][1][0, 0] * gains[1]
    w3 = p['conv_w'][2][0, 0] * gains[2]
    b1 = p['bn_beta'][0].reshape(1, 1)
    b2 = p['bn_beta'][1].reshape(1, 1)
    b3 = p['bn_beta'][2].reshape(1, 1)

    edge3 = edge_index.reshape(2, NCHUNK, CHUNK)
    edge3p = jnp.pad(edge3, ((0, 0), (0, NS * 80 - NCHUNK), (0, 0)))

    deg = _sc_degrees(edge3p)
    deg_out = deg[0].reshape(NPAD, 1)
    deg_in = deg[1].reshape(NPAD, 1)

    m1 = _conv1_matrix(w1)
    out1p = _tc_conv1(e_feat, m1, b1)
    out1d = (out1p.reshape(16, 400, 25, 16)
             .transpose(3, 1, 0, 2).reshape(16, 400, 400))
    wp = p['fc_e_W'].reshape(25, 25, HID).transpose(1, 0, 2)
    he2 = _tc_head(out1d, w2, b2, w3, b3, wp,
                   p['fc_e_b'].reshape(1, HID), deg_out[0:N_E])
    hm2 = _tc_fcm(m_feat, p['fc_m_W'], p['fc_m_b'].reshape(1, HID),
                  deg_out[N_E:N])
    hn0 = jnp.concatenate([he2, hm2], axis=1)

    z128 = jnp.zeros((NPAD, 128), jnp.float32)

    agg1 = _sc_aggregate(hn0.reshape(2 * N, 128), edge3p, z128, 128)
    hn1 = _tc_layer(agg1, deg_in, deg_out,
                    p['gc_W'][0].reshape(2, 128, HID), p['gc_b'][0].reshape(1, HID))
    agg2 = _sc_aggregate(hn1.reshape(2 * N, 128), edge3p, z128, 128)
    hn2 = _tc_layer(agg2, deg_in, deg_out,
                    p['gc_W'][1].reshape(2, 128, HID), p['gc_b'][1].reshape(1, HID))
    agg3 = _sc_aggregate(hn2.reshape(2 * N, 128), edge3p, z128, 128)
    return _tc_final(agg3, deg_in, p['gc_W'][2].reshape(2, 128, 16),
                     p['gc_b'][2].reshape(1, 16))


# speedup vs baseline: 10.9554x; 1.0015x over previous
import functools

import jax
import jax.numpy as jnp
from jax import lax
from jax.experimental import pallas as pl
from jax.experimental.pallas import tpu as pltpu
import jax.experimental.pallas.tpu_sc as plsc

N_E = 16
N_M = 9984
N = 10000
NPAD = 10240
E = 160000
HID = 256
NC, NS = 2, 16
CHUNK = 128
NCHUNK = E // CHUNK


def _sc_degrees(edge3p):
    mesh = plsc.VectorSubcoreMesh(core_axis_name="c", subcore_axis_name="s")
    NCB = 80

    @functools.partial(
        pl.kernel,
        out_type=jax.ShapeDtypeStruct((NC, NPAD), jnp.float32),
        mesh=mesh,
        scratch_types=[
            pltpu.VMEM_SHARED((NPAD,), jnp.float32),
            pltpu.VMEM((NCB, CHUNK), jnp.int32),
            pltpu.VMEM((CHUNK,), jnp.float32),
            pltpu.VMEM((640,), jnp.float32),
            pltpu.SemaphoreType.DMA,
        ],
    )
    def k(edge_hbm, out_hbm, hist_sh, idx_v, ones_v, z_v, sem):
        c = lax.axis_index("c")
        s = lax.axis_index("s")
        for i in range(CHUNK // 16):
            ones_v[pl.ds(i * 16, 16)] = jnp.full((16,), 1.0, jnp.float32)
        for i in range(640 // 16):
            z_v[pl.ds(i * 16, 16)] = jnp.zeros((16,), jnp.float32)
        pltpu.sync_copy(z_v, hist_sh.at[pl.ds(s * 640, 640)])
        cstart = s * NCB
        n = jnp.where(s == NS - 1, NCHUNK - (NS - 1) * NCB, NCB)
        pltpu.sync_copy(edge_hbm.at[c, pl.ds(cstart, NCB)], idx_v)
        plsc.subcore_barrier()

        def fire(t, carry):
            @pl.when(t < n)
            def _():
                pltpu.async_copy(ones_v, hist_sh.at[idx_v.at[t]], sem,
                                 add=True)
            return carry

        def drain(t, carry):
            @pl.when(t < n)
            def _():
                pltpu.make_async_copy(ones_v, hist_sh.at[idx_v.at[t]],
                                      sem).wait()
            return carry

        lax.fori_loop(0, NCB, fire, 0)
        lax.fori_loop(0, NCB, drain, 0)
        plsc.subcore_barrier()
        pltpu.sync_copy(hist_sh.at[pl.ds(s * 640, 640)],
                        out_hbm.at[c, pl.ds(s * 640, 640)])

    return k(edge3p)


def _sc_aggregate(table, edge3, zeros, fh):
    mesh = plsc.VectorSubcoreMesh(core_axis_name="c", subcore_axis_name="s")

    NCB = 80

    @functools.partial(
        pl.kernel,
        out_type=jax.ShapeDtypeStruct((NC, NPAD, fh), jnp.float32),
        mesh=mesh,
        scratch_types=[
            pltpu.VMEM_SHARED((NPAD, fh), jnp.float32),
            pltpu.VMEM((NCB // 2, CHUNK), jnp.int32),
            pltpu.VMEM((NCB // 2, CHUNK), jnp.int32),
            pltpu.VMEM((CHUNK, fh), jnp.float32),
            pltpu.VMEM((CHUNK, fh), jnp.float32),
            pltpu.SemaphoreType.DMA,
            pltpu.SemaphoreType.DMA,
        ],
    )
    def k(table_hbm, edge_hbm, zero_hbm, out_hbm,
          acc_sh, src_v, dst_v, rows0, rows1, sem0, sem1):
        c = lax.axis_index("c")
        s = lax.axis_index("s")
        pltpu.sync_copy(zero_hbm.at[pl.ds(s * 640, 640)],
                        acc_sh.at[pl.ds(s * 640, 640)])
        cstart = s * NCB
        ntot = jnp.where(s == NS - 1, NCHUNK - (NS - 1) * NCB, NCB)
        off = c * N
        HB = NCB // 2
        plsc.subcore_barrier()

        def gather(t, buf, sem):
            pltpu.async_copy(table_hbm.at[src_v.at[t]], buf, sem)

        def gwait(t, buf, sem):
            pltpu.make_async_copy(table_hbm.at[src_v.at[t]], buf, sem).wait()

        def scatter(t, buf):
            pltpu.sync_copy(buf, acc_sh.at[dst_v.at[t]], add=True)

        for ph in range(2):
            n = jnp.clip(ntot - ph * HB, 0, HB)

            @pl.when(n > 0)
            def _():
                pltpu.sync_copy(edge_hbm.at[0, pl.ds(cstart + ph * HB, HB)],
                                src_v)
                pltpu.sync_copy(edge_hbm.at[1, pl.ds(cstart + ph * HB, HB)],
                                dst_v)

                def add_off(t, carry):
                    for i in range(CHUNK // 16):
                        src_v[t, pl.ds(i * 16, 16)] = (
                            src_v[t, pl.ds(i * 16, 16)] + off)
                    return carry

                lax.fori_loop(0, n, add_off, 0)
                gather(0, rows0, sem0)

                @pl.when(n > 1)
                def _():
                    gather(1, rows1, sem1)

                def body(i, carry):
                    t0 = 2 * i
                    t1 = 2 * i + 1

                    @pl.when(t0 < n)
                    def _():
                        gwait(t0, rows0, sem0)
                        scatter(t0, rows0)

                        @pl.when(t0 + 2 < n)
                        def _():
                            gather(t0 + 2, rows0, sem0)

                    @pl.when(t1 < n)
                    def _():
                        gwait(t1, rows1, sem1)
                        scatter(t1, rows1)

                        @pl.when(t1 + 2 < n)
                        def _():
                            gather(t1 + 2, rows1, sem1)

                    return carry

                lax.fori_loop(0, HB // 2, body, 0)

        plsc.subcore_barrier()
        pltpu.sync_copy(acc_sh.at[pl.ds(s * 640, 640)],
                        out_hbm.at[c, pl.ds(s * 640, 640)])

    return k(table, edge3, zeros)


def _lrelu(x):
    return jnp.where(x >= 0, x, 0.1 * x)


import numpy as _np

_C1MASK = (_np.arange(512)[:, None] // 4 ==
           _np.arange(128)[None, :]).astype(_np.float32)
_C1BSEL = _np.arange(512) % 4


def _conv1_matrix(w1):
    return jnp.stack([_C1MASK * w1[a][_C1BSEL][:, None]
                      for a in range(4)]).astype(jnp.bfloat16)


def _tc_conv1(e_feat, m1, beta1):
    def body(b_ref, x_ref, m_ref, o_ref):
        xb = x_ref[0].astype(jnp.bfloat16)
        m = m_ref[...]
        blocks = []
        for jb in range(4):
            kw = 512 if jb < 3 else 64
            nw = 128 if jb < 3 else 16
            xs = xb[:, 512 * jb:512 * jb + kw]
            acc = jnp.zeros((400, nw), jnp.float32)
            for a in range(4):
                t = jnp.dot(xs, m[a, 0:kw, 0:nw],
                            preferred_element_type=jnp.float32)
                acc = acc + t.reshape(400, 4, nw)[:, a, :]
            blocks.append(acc)
        res = jnp.concatenate(blocks, axis=1)
        o_ref[0] = _lrelu(res + b_ref[0, 0])

    return pl.pallas_call(
        body,
        grid=(16,),
        in_specs=[
            pl.BlockSpec(memory_space=pltpu.SMEM),
            pl.BlockSpec((1, 1600, 1600), lambda img: (img, 0, 0)),
            pl.BlockSpec((4, 512, 128), lambda img: (0, 0, 0)),
        ],
        out_specs=pl.BlockSpec((1, 400, 400), lambda img: (img, 0, 0)),
        out_shape=jax.ShapeDtypeStruct((16, 400, 400), jnp.float32),
    )(beta1, e_feat, m1)


def _tc_head(out1d, w2, beta2, w3, beta3, wp, fce_b, deg_e):
    def body(w2_ref, b2_ref, w3_ref, b3_ref, x_ref, wp_ref, fb_ref, de_ref,
             o_ref):
        out2 = []
        for u2 in range(4):
            acc = jnp.zeros((100, 400), jnp.float32)
            for a in range(4):
                for b in range(4):
                    xs = x_ref[4 * u2 + b].reshape(100, 4, 400)[:, a, :]
                    acc = acc + xs * w2_ref[a, b]
            out2.append(_lrelu(acc + b2_ref[0, 0]))
        acc3 = jnp.zeros((25, 400), jnp.float32)
        for a in range(4):
            for b in range(4):
                xs = out2[b].reshape(25, 4, 400)[:, a, :]
                acc3 = acc3 + xs * w3_ref[a, b]
        out3 = _lrelu(acc3 + b3_ref[0, 0])
        o3r = out3.T.reshape(16, 25, 25)
        he = jnp.zeros((16, HID), jnp.float32)
        for t3 in range(25):
            he = he + jnp.dot(o3r[:, t3, :], wp_ref[t3],
                              preferred_element_type=jnp.float32)
        dinv = lax.rsqrt(jnp.maximum(de_ref[...], 1.0))
        he = (he + fb_ref[...]) * dinv
        o_ref[0] = he[:, 0:128]
        o_ref[1] = he[:, 128:256]

    return pl.pallas_call(
        body,
        in_specs=[
            pl.BlockSpec(memory_space=pltpu.SMEM),
            pl.BlockSpec(memory_space=pltpu.SMEM),
            pl.BlockSpec(memory_space=pltpu.SMEM),
            pl.BlockSpec(memory_space=pltpu.SMEM),
            pl.BlockSpec((16, 400, 400), lambda: (0, 0, 0)),
            pl.BlockSpec((25, 25, HID), lambda: (0, 0, 0)),
            pl.BlockSpec((1, HID), lambda: (0, 0)),
            pl.BlockSpec((16, 1), lambda: (0, 0)),
        ],
        out_specs=pl.BlockSpec((2, 16, 128), lambda: (0, 0, 0)),
        out_shape=jax.ShapeDtypeStruct((2, 16, 128), jnp.float32),
    )(w2, beta2, w3, beta3, out1d, wp, fce_b, deg_e)


def _tc_fcm(m_feat, wm, bm, deg_m):
    def body(x_ref, w_ref, b_ref, d_ref, o_ref):
        z = jnp.dot(x_ref[...], w_ref[...],
                    preferred_element_type=jnp.float32) + b_ref[...]
        dinv = lax.rsqrt(jnp.maximum(d_ref[...], 1.0))
        o_ref[0] = z * dinv

    return pl.pallas_call(
        body,
        grid=(2, 16),
        in_specs=[
            pl.BlockSpec((624, HID), lambda h, r: (r, 0)),
            pl.BlockSpec((HID, 128), lambda h, r: (0, h)),
            pl.BlockSpec((1, 128), lambda h, r: (0, h)),
            pl.BlockSpec((624, 1), lambda h, r: (r, 0)),
        ],
        out_specs=pl.BlockSpec((1, 624, 128), lambda h, r: (h, r, 0)),
        out_shape=jax.ShapeDtypeStruct((2, N_M, 128), jnp.float32),
    )(m_feat, wm, bm, deg_m)


def _tc_layer(agg, deg_in, deg_out, wr, b):
    def body(a_ref, di_ref, do_ref, w_ref, b_ref, o_ref):
        din = lax.rsqrt(jnp.maximum(di_ref[...], 1.0))
        a = a_ref[...]
        z = (jnp.dot(a[0] * din, w_ref[0], preferred_element_type=jnp.float32)
             + jnp.dot(a[1] * din, w_ref[1], preferred_element_type=jnp.float32)
             + b_ref[...])
        dout = lax.rsqrt(jnp.maximum(do_ref[...], 1.0))
        o_ref[0] = jnp.maximum(z, 0.0) * dout

    return pl.pallas_call(
        body,
        grid=(2, 25),
        in_specs=[
            pl.BlockSpec((2, 400, 128), lambda h, r: (0, r, 0)),
            pl.BlockSpec((400, 1), lambda h, r: (r, 0)),
            pl.BlockSpec((400, 1), lambda h, r: (r, 0)),
            pl.BlockSpec((2, 128, 128), lambda h, r: (0, 0, h)),
            pl.BlockSpec((1, 128), lambda h, r: (0, h)),
        ],
        out_specs=pl.BlockSpec((1, 400, 128), lambda h, r: (h, r, 0)),
        out_shape=jax.ShapeDtypeStruct((2, N, 128), jnp.float32),
    )(agg, deg_in, deg_out, wr, b)


def _tc_final(agg, deg_in, w3r, b3):
    def body(a_ref, di_ref, w3_ref, b_ref, o_ref):
        din = lax.rsqrt(jnp.maximum(di_ref[...], 1.0))
        a = a_ref[...]
        o_ref[...] = (
            jnp.dot(a[0] * din, w3_ref[0], preferred_element_type=jnp.float32)
            + jnp.dot(a[1] * din, w3_ref[1], preferred_element_type=jnp.float32)
            + b_ref[...])

    return pl.pallas_call(
        body,
        grid=(25,),
        in_specs=[
            pl.BlockSpec((2, 400, 128), lambda r: (0, r, 0)),
            pl.BlockSpec((400, 1), lambda r: (r, 0)),
            pl.BlockSpec((2, 128, 16), lambda r: (0, 0, 0)),
            pl.BlockSpec((1, 16), lambda r: (0, 0)),
        ],
        out_specs=pl.BlockSpec((400, 16), lambda r: (r, 0)),
        out_shape=jax.ShapeDtypeStruct((N, 16), jnp.float32),
    )(agg, deg_in, w3r, b3)


def kernel(e_feat, m_feat, edge_index, params):
    p = params
    gains = [p['bn_gamma'][i][0] / jnp.sqrt(jnp.float32(1.0 + 1e-5))
             for i in range(3)]
    w1 = p['conv_w'][0][0, 0] * gains[0]
    w2 = p['conv_w'][1][0, 0] * gains[1]
    w3 = p['conv_w'][2][0, 0] * gains[2]
    b1 = p['bn_beta'][0].reshape(1, 1)
    b2 = p['bn_beta'][1].reshape(1, 1)
    b3 = p['bn_beta'][2].reshape(1, 1)

    edge3 = edge_index.reshape(2, NCHUNK, CHUNK)
    edge3p = jnp.pad(edge3, ((0, 0), (0, NS * 80 - NCHUNK), (0, 0)))

    deg = _sc_degrees(edge3p)
    deg_out = deg[0].reshape(NPAD, 1)
    deg_in = deg[1].reshape(NPAD, 1)

    m1 = _conv1_matrix(w1)
    out1p = _tc_conv1(e_feat, m1, b1)
    out1d = (out1p.reshape(16, 400, 25, 16)
             .transpose(3, 1, 0, 2).reshape(16, 400, 400))
    wp = p['fc_e_W'].reshape(25, 25, HID).transpose(1, 0, 2)
    he2 = _tc_head(out1d, w2, b2, w3, b3, wp,
                   p['fc_e_b'].reshape(1, HID), deg_out[0:N_E])
    hm2 = _tc_fcm(m_feat, p['fc_m_W'], p['fc_m_b'].reshape(1, HID),
                  deg_out[N_E:N])
    hn0 = jnp.concatenate([he2, hm2], axis=1)

    z128 = jnp.zeros((NPAD, 128), jnp.float32)

    agg1 = _sc_aggregate(hn0.reshape(2 * N, 128), edge3p, z128, 128)
    hn1 = _tc_layer(agg1, deg_in, deg_out,
                    p['gc_W'][0].reshape(2, 128, HID), p['gc_b'][0].reshape(1, HID))
    agg2 = _sc_aggregate(hn1.reshape(2 * N, 128), edge3p, z128, 128)
    hn2 = _tc_layer(agg2, deg_in, deg_out,
                    p['gc_W'][1].reshape(2, 128, HID), p['gc_b'][1].reshape(1, HID))
    agg3 = _sc_aggregate(hn2.reshape(2 * N, 128), edge3p, z128, 128)
    return _tc_final(agg3, deg_in, p['gc_W'][2].reshape(2, 128, 16),
                     p['gc_b'][2].reshape(1, 16))
